# Initial kernel scaffold; baseline (speedup 1.0000x reference)
import jax
import jax.numpy as jnp
from jax.experimental import pallas as pl


def kernel(x, L_rows, L_cols, L_vals, W):
    raise NotImplementedError("write your pallas kernel here")


# speedup vs baseline: 1.8878x; 1.8878x over previous
import functools

import jax
import jax.numpy as jnp
from jax import lax
from jax.experimental import pallas as pl
from jax.experimental.pallas import tpu as pltpu
from jax.experimental.pallas import tpu_sc as plsc

B = 16
M = 10000
FIN = 128
NK = 6
FOUT = 64
E = 320000

NC = 2
NS = 16
SLABS = B // NC
K = 128
NCHUNK = -(-(E // NS) // K)
EPT = NCHUNK * K
EP = EPT * NS
STRIPE = 640
RP = 64
NRP = STRIPE // RP
PARTIAL = 16
LANES = 16


def _make_spmm(subtract: bool, alpha: float):
    mesh = plsc.VectorSubcoreMesh(core_axis_name="c", subcore_axis_name="s")

    @functools.partial(
        pl.kernel,
        out_type=jax.ShapeDtypeStruct((B * M, FIN), jnp.float32),
        mesh=mesh,
        scratch_types=[
            pltpu.VMEM((K,), jnp.int32),
            pltpu.VMEM((K,), jnp.int32),
            pltpu.VMEM((K,), jnp.float32),
            pltpu.VMEM((K, FIN), jnp.float32),
            pltpu.VMEM((RP, FIN), jnp.float32),
            pltpu.VMEM((RP, FIN), jnp.float32),
            pltpu.VMEM_SHARED((M, FIN), jnp.float32),
            pltpu.SemaphoreType.DMA,
        ],
    )
    def spmm(x_hbm, zprev_hbm, rows_hbm, cols_hbm, vals_hbm, y_hbm,
             ridx_v, cidx_v, vals_v, gath_v, tmp_v, tmp2_v,
             acc_sh, sem):
        c = lax.axis_index("c")
        s = lax.axis_index("s")
        edge_base = s * EPT
        stripe_lo = s * STRIPE

        zeros16 = jnp.zeros((LANES,), jnp.float32)

        def slab_body(bi, _):
            b = c * SLABS + bi
            xbase = b * M

            def zero_row(j, _):
                for cc in range(FIN // LANES):
                    tmp_v[j, pl.ds(cc * LANES, LANES)] = zeros16
                return 0

            lax.fori_loop(0, RP, zero_row, 0)
            for p in range(NRP):
                rlo = stripe_lo + p * RP

                @pl.when(rlo + RP <= M)
                def _():
                    pltpu.sync_copy(tmp_v, acc_sh.at[pl.ds(rlo, RP)])

                @pl.when(jnp.logical_and(rlo < M, rlo + RP > M))
                def _():
                    pltpu.sync_copy(tmp_v.at[pl.ds(0, PARTIAL)],
                                    acc_sh.at[pl.ds(rlo, PARTIAL)])

            plsc.subcore_barrier()

            def chunk_body(i, _):
                off = edge_base + i * K
                pltpu.sync_copy(rows_hbm.at[pl.ds(off, K)], ridx_v)
                pltpu.sync_copy(cols_hbm.at[pl.ds(off, K)], cidx_v)
                pltpu.sync_copy(vals_hbm.at[pl.ds(off, K)], vals_v)

                for j in range(K // LANES):
                    sl = pl.ds(j * LANES, LANES)
                    cidx_v[sl] = cidx_v[sl] + xbase

                pltpu.async_copy(x_hbm.at[cidx_v], gath_v, sem).wait()

                def scale_group(g, _):
                    vv = vals_v[pl.ds(g * LANES, LANES)] * alpha
                    for l in range(LANES):
                        v = vv[l]
                        j = g * LANES + l
                        for cc in range(FIN // LANES):
                            sl = pl.ds(cc * LANES, LANES)
                            gath_v[j, sl] = gath_v[j, sl] * v
                    return 0

                lax.fori_loop(0, K // LANES, scale_group, 0)

                pltpu.sync_copy(gath_v, acc_sh.at[ridx_v], add=True)
                return 0

            lax.fori_loop(0, NCHUNK, chunk_body, 0)
            plsc.subcore_barrier()

            for p in range(NRP):
                rlo = stripe_lo + p * RP

                @pl.when(rlo + RP <= M)
                def _():
                    pltpu.sync_copy(acc_sh.at[pl.ds(rlo, RP)], tmp_v)
                    if subtract:
                        pltpu.sync_copy(zprev_hbm.at[pl.ds(xbase + rlo, RP)],
                                        tmp2_v)

                        def sub_row(j, _):
                            for cc in range(FIN // LANES):
                                sl = pl.ds(cc * LANES, LANES)
                                tmp_v[j, sl] = tmp_v[j, sl] - tmp2_v[j, sl]
                            return 0

                        lax.fori_loop(0, RP, sub_row, 0)
                    pltpu.sync_copy(tmp_v, y_hbm.at[pl.ds(xbase + rlo, RP)])

                @pl.when(jnp.logical_and(rlo < M, rlo + RP > M))
                def _():
                    pltpu.sync_copy(acc_sh.at[pl.ds(rlo, PARTIAL)],
                                    tmp_v.at[pl.ds(0, PARTIAL)])
                    if subtract:
                        pltpu.sync_copy(
                            zprev_hbm.at[pl.ds(xbase + rlo, PARTIAL)],
                            tmp2_v.at[pl.ds(0, PARTIAL)])

                        def sub_row_p(j, _):
                            for cc in range(FIN // LANES):
                                sl = pl.ds(cc * LANES, LANES)
                                tmp_v[j, sl] = tmp_v[j, sl] - tmp2_v[j, sl]
                            return 0

                        lax.fori_loop(0, PARTIAL, sub_row_p, 0)
                    pltpu.sync_copy(tmp_v.at[pl.ds(0, PARTIAL)],
                                    y_hbm.at[pl.ds(xbase + rlo, PARTIAL)])
            return 0

        lax.fori_loop(0, SLABS, slab_body, 0)

    if subtract:
        return spmm

    def spmm_nosub(x_flat, rows, cols, vals):
        return spmm(x_flat, x_flat, rows, cols, vals)

    return spmm_nosub


def _proj_body(z0, z1, z2, z3, z4, z5, w_ref, o_ref):
    acc = jnp.zeros(o_ref.shape, jnp.float32)
    for k, z in enumerate((z0, z1, z2, z3, z4, z5)):
        acc += jnp.dot(z[...], w_ref[k], preferred_element_type=jnp.float32)
    o_ref[...] = acc


def _projection(zs, wk):
    BM = 1000
    grid = ((B * M) // BM,)
    zspec = pl.BlockSpec((BM, FIN), lambda i: (i, 0))
    return pl.pallas_call(
        _proj_body,
        grid=grid,
        in_specs=[zspec] * NK + [pl.BlockSpec((NK, FIN, FOUT), lambda i: (0, 0, 0))],
        out_specs=pl.BlockSpec((BM, FOUT), lambda i: (i, 0)),
        out_shape=jax.ShapeDtypeStruct((B * M, FOUT), jnp.float32),
    )(*zs, wk)


def kernel(x, L_rows, L_cols, L_vals, W):
    x_flat = x.reshape(B * M, FIN)
    pad = EP - E
    rows_p = jnp.concatenate([L_rows, jnp.zeros((pad,), jnp.int32)])
    cols_p = jnp.concatenate([L_cols, jnp.zeros((pad,), jnp.int32)])
    vals_p = jnp.concatenate([L_vals, jnp.zeros((pad,), jnp.float32)])
    wk = W.reshape(FIN, NK, FOUT).transpose(1, 0, 2)

    spmm1 = _make_spmm(subtract=False, alpha=1.0)
    spmm2 = _make_spmm(subtract=True, alpha=2.0)

    z0 = x_flat
    z1 = spmm1(z0, rows_p, cols_p, vals_p)
    z2 = spmm2(z1, z0, rows_p, cols_p, vals_p)
    z3 = spmm2(z2, z1, rows_p, cols_p, vals_p)
    z4 = spmm2(z3, z2, rows_p, cols_p, vals_p)
    z5 = spmm2(z4, z3, rows_p, cols_p, vals_p)

    out = _projection((z0, z1, z2, z3, z4, z5), wk)
    return out.reshape(B, M, FOUT)


# speedup vs baseline: 2.8348x; 1.5017x over previous
import functools

import jax
import jax.numpy as jnp
from jax import lax
from jax.experimental import pallas as pl
from jax.experimental.pallas import tpu as pltpu
from jax.experimental.pallas import tpu_sc as plsc

B = 16
M = 10000
FIN = 128
NK = 6
FOUT = 64
E = 320000

NC = 2
NS = 16
NSL = B // NC
K = 128
D = 2
DG = 1
NCHUNK = 158
EPT = NCHUNK * K
EP = EPT * NS
STRIPE = 640
RP = 64
NRP = STRIPE // RP
PARTIAL = 16
LANES = 16


def _make_spmm(subtract: bool, alpha: float):
    mesh = plsc.VectorSubcoreMesh(core_axis_name="c", subcore_axis_name="s")

    @functools.partial(
        pl.kernel,
        out_type=jax.ShapeDtypeStruct((B * M, FIN), jnp.float32),
        mesh=mesh,
        scratch_types=(
            [pltpu.VMEM((K,), jnp.int32) for _ in range(D)]
            + [pltpu.VMEM((K,), jnp.int32) for _ in range(D)]
            + [pltpu.VMEM((K,), jnp.float32) for _ in range(D)]
            + [pltpu.VMEM((K, FIN), jnp.float32) for _ in range(D)]
            + [
                pltpu.VMEM((RP, FIN), jnp.float32),
                pltpu.VMEM((RP, FIN), jnp.float32),
                pltpu.VMEM_SHARED((M, FIN), jnp.float32),
            ]
            + [pltpu.SemaphoreType.DMA for _ in range(2 * D)]
        ),
    )
    def spmm(x_hbm, zprev_hbm, rows_hbm, cols_hbm, vals_hbm, y_hbm, *refs):
        ridx = refs[0:D]
        cidx = refs[D:2 * D]
        vals = refs[2 * D:3 * D]
        gath = refs[3 * D:4 * D]
        tmp_v = refs[4 * D]
        tmp2_v = refs[4 * D + 1]
        acc_sh = refs[4 * D + 2]
        isem = refs[4 * D + 3:4 * D + 3 + D]
        gsem = refs[4 * D + 3 + D:4 * D + 3 + 2 * D]

        c = lax.axis_index("c")
        s = lax.axis_index("s")
        edge_base = s * EPT
        stripe_lo = s * STRIPE

        zeros16 = jnp.zeros((LANES,), jnp.float32)

        def fire_idx(j, r):
            off = edge_base + j * K
            pltpu.async_copy(rows_hbm.at[pl.ds(off, K)], ridx[r], isem[r])
            pltpu.async_copy(cols_hbm.at[pl.ds(off, K)], cidx[r], isem[r])
            pltpu.async_copy(vals_hbm.at[pl.ds(off, K)], vals[r], isem[r])

        def wait_idx(j, r):
            off = edge_base + j * K
            pltpu.make_async_copy(rows_hbm.at[pl.ds(off, K)], ridx[r], isem[r]).wait()
            pltpu.make_async_copy(cols_hbm.at[pl.ds(off, K)], cidx[r], isem[r]).wait()
            pltpu.make_async_copy(vals_hbm.at[pl.ds(off, K)], vals[r], isem[r]).wait()

        def adjust_and_fire_gather(r, xbase):
            for g in range(K // LANES):
                sl = pl.ds(g * LANES, LANES)
                cidx[r][sl] = cidx[r][sl] + xbase
            pltpu.async_copy(x_hbm.at[cidx[r]], gath[r], gsem[r])

        def wait_gather(r):
            pltpu.make_async_copy(x_hbm.at[cidx[r]], gath[r], gsem[r]).wait()

        def slab_body(bi, _):
            b = c * NSL + bi
            xbase = b * M

            def zero_row(q, _):
                for ccx in range(FIN // LANES):
                    tmp_v[q, pl.ds(ccx * LANES, LANES)] = zeros16
                return 0

            lax.fori_loop(0, RP, zero_row, 0)
            for p in range(NRP):
                rlo = stripe_lo + p * RP

                @pl.when(rlo + RP <= M)
                def _():
                    pltpu.sync_copy(tmp_v, acc_sh.at[pl.ds(rlo, RP)])

                @pl.when(jnp.logical_and(rlo < M, rlo + RP > M))
                def _():
                    pltpu.sync_copy(tmp_v.at[pl.ds(0, PARTIAL)],
                                    acc_sh.at[pl.ds(rlo, PARTIAL)])

            plsc.subcore_barrier()

            for r in range(D):
                fire_idx(r, r)
            for r in range(DG):
                wait_idx(r, r)
                adjust_and_fire_gather(r, xbase)

            def ring_body(i, _):
                for r in range(D):
                    j = i * D + r

                    @pl.when(j + DG < NCHUNK)
                    def _():
                        rg = (r + DG) % D
                        wait_idx(j + DG, rg)
                        adjust_and_fire_gather(rg, xbase)

                    wait_gather(r)

                    def scale_group(g, _):
                        vv = vals[r][pl.ds(g * LANES, LANES)] * alpha
                        for l in range(LANES):
                            v = vv[l]
                            q = g * LANES + l
                            for ccx in range(FIN // LANES):
                                sl2 = pl.ds(ccx * LANES, LANES)
                                gath[r][q, sl2] = gath[r][q, sl2] * v
                        return 0

                    lax.fori_loop(0, K // LANES, scale_group, 0)

                    pltpu.sync_copy(gath[r], acc_sh.at[ridx[r]], add=True)

                    @pl.when(j + D < NCHUNK)
                    def _():
                        fire_idx(j + D, r)
                return 0

            lax.fori_loop(0, NCHUNK // D, ring_body, 0)
            plsc.subcore_barrier()

            for p in range(NRP):
                rlo = stripe_lo + p * RP

                @pl.when(rlo + RP <= M)
                def _():
                    pltpu.sync_copy(acc_sh.at[pl.ds(rlo, RP)], tmp_v)
                    if subtract:
                        pltpu.sync_copy(zprev_hbm.at[pl.ds(xbase + rlo, RP)],
                                        tmp2_v)

                        def sub_row(q, _):
                            for ccx in range(FIN // LANES):
                                sl2 = pl.ds(ccx * LANES, LANES)
                                tmp_v[q, sl2] = tmp_v[q, sl2] - tmp2_v[q, sl2]
                            return 0

                        lax.fori_loop(0, RP, sub_row, 0)
                    pltpu.sync_copy(tmp_v, y_hbm.at[pl.ds(xbase + rlo, RP)])

                @pl.when(jnp.logical_and(rlo < M, rlo + RP > M))
                def _():
                    pltpu.sync_copy(acc_sh.at[pl.ds(rlo, PARTIAL)],
                                    tmp_v.at[pl.ds(0, PARTIAL)])
                    if subtract:
                        pltpu.sync_copy(
                            zprev_hbm.at[pl.ds(xbase + rlo, PARTIAL)],
                            tmp2_v.at[pl.ds(0, PARTIAL)])

                        def sub_row_p(q, _):
                            for ccx in range(FIN // LANES):
                                sl2 = pl.ds(ccx * LANES, LANES)
                                tmp_v[q, sl2] = tmp_v[q, sl2] - tmp2_v[q, sl2]
                            return 0

                        lax.fori_loop(0, PARTIAL, sub_row_p, 0)
                    pltpu.sync_copy(tmp_v.at[pl.ds(0, PARTIAL)],
                                    y_hbm.at[pl.ds(xbase + rlo, PARTIAL)])
            return 0

        lax.fori_loop(0, NSL, slab_body, 0)

    if subtract:
        return spmm

    def spmm_nosub(x_flat, rows, cols, vals):
        return spmm(x_flat, x_flat, rows, cols, vals)

    return spmm_nosub


def _proj_body(z0, z1, z2, z3, z4, z5, w_ref, o_ref):
    acc = jnp.zeros(o_ref.shape, jnp.float32)
    for k, z in enumerate((z0, z1, z2, z3, z4, z5)):
        acc += jnp.dot(z[...], w_ref[k], preferred_element_type=jnp.float32)
    o_ref[...] = acc


def _projection(zs, wk):
    BM = 1000
    grid = ((B * M) // BM,)
    zspec = pl.BlockSpec((BM, FIN), lambda i: (i, 0))
    return pl.pallas_call(
        _proj_body,
        grid=grid,
        in_specs=[zspec] * NK + [pl.BlockSpec((NK, FIN, FOUT), lambda i: (0, 0, 0))],
        out_specs=pl.BlockSpec((BM, FOUT), lambda i: (i, 0)),
        out_shape=jax.ShapeDtypeStruct((B * M, FOUT), jnp.float32),
    )(*zs, wk)


def kernel(x, L_rows, L_cols, L_vals, W):
    x_flat = x.reshape(B * M, FIN)
    pad = EP - E
    rows_p = jnp.concatenate([L_rows, jnp.zeros((pad,), jnp.int32)])
    cols_p = jnp.concatenate([L_cols, jnp.zeros((pad,), jnp.int32)])
    vals_p = jnp.concatenate([L_vals, jnp.zeros((pad,), jnp.float32)])
    wk = W.reshape(FIN, NK, FOUT).transpose(1, 0, 2)

    spmm1 = _make_spmm(subtract=False, alpha=1.0)
    spmm2 = _make_spmm(subtract=True, alpha=2.0)

    z0 = x_flat
    z1 = spmm1(z0, rows_p, cols_p, vals_p)
    z2 = spmm2(z1, z0, rows_p, cols_p, vals_p)
    z3 = spmm2(z2, z1, rows_p, cols_p, vals_p)
    z4 = spmm2(z3, z2, rows_p, cols_p, vals_p)
    z5 = spmm2(z4, z3, rows_p, cols_p, vals_p)

    out = _projection((z0, z1, z2, z3, z4, z5), wk)
    return out.reshape(B, M, FOUT)
